# baseline (speedup 1.0000x reference)
import functools
import jax
import jax.numpy as jnp
from jax import lax
from jax.experimental import pallas as pl
from jax.experimental.pallas import tpu as pltpu
from jax.experimental.pallas import tpu_sc as plsc

D_M = 64
PAD_W = 128
SCALE = 8.0
NW = 32
CW = 128
NBUF = 4
AHEAD = 2
SLAB = 32


def _gather_call(R, C):
    RW = R // NW
    NS = RW // SLAB
    OFF_B = C - CW
    mesh = plsc.VectorSubcoreMesh(core_axis_name="c", subcore_axis_name="s")

    @functools.partial(
        pl.kernel,
        mesh=mesh,
        out_type=jax.ShapeDtypeStruct((R, C, PAD_W), jnp.float32),
        scratch_types=[
            pltpu.VMEM((2, SLAB, CW), jnp.int32),
            pltpu.VMEM((2, SLAB, CW), jnp.int32),
            pltpu.VMEM((NBUF, CW, PAD_W), jnp.float32),
            pltpu.SemaphoreType.DMA((2,)),
            pltpu.SemaphoreType.DMA((2,)),
            pltpu.SemaphoreType.DMA((NBUF,)),
            pltpu.SemaphoreType.DMA((NBUF,)),
        ],
    )
    def body(xa_hbm, xb_hbm, lutp_hbm, out_hbm, iA, iB, bufs,
             isemA, isemB, gsems, osems):
        wid = lax.axis_index("s") * 2 + lax.axis_index("c")
        rbase = wid * RW

        def stage(s, sem_wait=False):
            rows = pl.ds(rbase + s * SLAB, SLAB)
            argsA = (xa_hbm.at[rows], iA.at[s % 2], isemA.at[s % 2])
            argsB = (xb_hbm.at[rows], iB.at[s % 2], isemB.at[s % 2])
            if sem_wait:
                pltpu.make_async_copy(*argsA).wait()
                pltpu.make_async_copy(*argsB).wait()
            else:
                pltpu.async_copy(*argsA)
                pltpu.async_copy(*argsB)

        def gather(c, b, start):
            r = c // 2
            sb = (r // SLAB) % 2
            rs = r % SLAB
            src = iA if b % 2 == 0 else iB
            args = (lutp_hbm.at[src.at[sb, rs]], bufs.at[b], gsems.at[b])
            if start:
                pltpu.async_copy(*args)
            else:
                pltpu.make_async_copy(*args).wait()

        def store(c, b, start):
            r = c // 2
            c0 = 0 if b % 2 == 0 else OFF_B
            args = (
                bufs.at[b],
                out_hbm.at[rbase + r, pl.ds(c0, CW)],
                osems.at[b],
            )
            if start:
                pltpu.async_copy(*args)
            else:
                pltpu.make_async_copy(*args).wait()

        stage(0)
        stage(1)
        stage(0, sem_wait=True)
        for c in range(AHEAD):
            gather(c, c % NBUF, start=True)

        NCH = 2 * RW

        def slab_loop(s, carry):
            @pl.when(s + 1 < NS)
            def _wait_next():
                stage(s + 1, sem_wait=True)

            def block(t, c1):
                for u in range(NBUF):
                    c = NBUF * t + u
                    cf = c + AHEAD
                    bf = (u + AHEAD) % NBUF

                    @pl.when(cf < NCH)
                    def _fire():
                        @pl.when(cf >= NBUF)
                        def _drain():
                            store(cf - NBUF, bf, start=False)

                        gather(cf, bf, start=True)

                    gather(c, u, start=False)
                    store(c, u, start=True)
                return c1

            nblk = 2 * SLAB // NBUF
            lax.fori_loop(s * nblk, (s + 1) * nblk, block, 0)

            @pl.when(s + 2 < NS)
            def _restage():
                stage(s + 2)

            return carry

        lax.fori_loop(0, NS, slab_loop, 0)

        for c in range(NCH - NBUF, NCH):
            store(c, c % NBUF, start=False)

    return body


def kernel(x, lut):
    xi = x.astype(jnp.int32)
    xa = xi[:, :CW]
    xb = xi[:, x.shape[1] - CW:]
    lutp = jnp.pad(lut * SCALE, ((0, 0), (0, PAD_W - D_M)))
    outp = _gather_call(x.shape[0], x.shape[1])(xa, xb, lutp)
    return outp[:, :, :D_M]
